# baseline (speedup 1.0000x reference)
import functools

import jax
import jax.numpy as jnp
from jax import lax
from jax.experimental import pallas as pl
from jax.experimental.pallas import tpu as pltpu
from jax.experimental.pallas import tpu_sc as plsc

BS = 64
NQ = 32768
K = NQ // 16
NBUCKET = 1024
NB_VREG = NBUCKET // 16


def _cent_body(x_ref, y_ref, box_ref, cent_ref):
    xb = x_ref[...]
    yb = y_ref[...]
    cx = box_ref[:, 0:1]
    cy = box_ref[:, 1:2]
    w = box_ref[:, 2:3]
    h = box_ref[:, 3:4]
    xmin = cx - w / 2.0
    ymin = cy - h / 2.0
    xmax = cx + w / 2.0
    ymax = cy + h / 2.0
    left = jnp.clip(xb - xmin, 0.0, 1.0)
    right = jnp.clip(xmax - xb, 0.0, 1.0)
    top = jnp.clip(yb - ymin, 0.0, 1.0)
    down = jnp.clip(ymax - yb, 0.0, 1.0)
    sx = (left + right) / 2.0
    dx = jnp.abs(left - right) / 2.0
    sy = (top + down) / 2.0
    dy = jnp.abs(top - down) / 2.0
    cxn = (sx - dx) / (sx + dx)
    cyn = (sy - dy) / (sy + dy)
    cent_ref[...] = jnp.sqrt(cxn * cyn)


def _mask_body(cent_ref, thr_ref, mask_ref):
    mask_ref[...] = cent_ref[...] > thr_ref[:, 0:1]


def _select_row(row_v, hist_v):
    lanes = lax.iota(jnp.int32, 16)
    ones = jnp.ones((16,), jnp.int32)
    zeros16 = jnp.zeros((16,), jnp.int32)

    prefix = jnp.int32(0)
    need = jnp.int32(K + 1)
    for p in range(3):
        shift = 20 - 10 * p

        def zero_body(j, _):
            hist_v[pl.ds(j * 16, 16)] = zeros16
            return 0

        lax.fori_loop(0, NB_VREG, zero_body, 0, unroll=8)

        pfx = prefix

        def hist_body(i, _):
            v = row_v[pl.ds(i * 16, 16)]
            u = lax.bitcast_convert_type(v, jnp.int32)
            u = jnp.where(v == v, u, 0)
            bucket = lax.shift_right_logical(u, shift) & (NBUCKET - 1)
            if p == 0:
                plsc.addupdate_scatter(hist_v, [bucket], ones)
            else:
                m = lax.shift_right_logical(u, shift + 10) == pfx
                plsc.addupdate_scatter(hist_v, [bucket], ones, mask=m)
            return 0

        lax.fori_loop(0, NQ // 16, hist_body, 0, unroll=4)

        def scan_body(t, carry):
            above, found, bsel, above_sel = carry
            j = NB_VREG - 1 - t
            h = hist_v[pl.ds(j * 16, 16)]
            csum = plsc.cumsum(h)
            total = jnp.max(csum)
            s = total - csum + h
            ge = (above + s) >= need
            cnt = jnp.max(plsc.all_reduce_population_count(ge))
            istar = cnt - 1
            h_at = jnp.max(jnp.where(lanes == istar, h, 0))
            s_at = jnp.max(jnp.where(lanes == istar, s, jnp.int32(-2**31)))
            hit = jnp.logical_and(cnt > 0, jnp.logical_not(found))
            bsel = jnp.where(hit, j * 16 + istar, bsel)
            above_sel = jnp.where(hit, above + s_at - h_at, above_sel)
            above = jnp.where(found | hit, above, above + total)
            return above, found | hit, bsel, above_sel

        _, _, bsel, above_sel = lax.fori_loop(
            0, NB_VREG, scan_body,
            (jnp.int32(0), jnp.bool_(False), jnp.int32(0), jnp.int32(0)))

        prefix = jnp.where(p == 0, bsel, (prefix << 10) | bsel)
        need = need - above_sel

    return jnp.broadcast_to(prefix << 0, (16,))


def _sc_select(cent_hbm, thr_hbm, row_v, hist_v, thr_v):
    cid = lax.axis_index("c")
    sid = lax.axis_index("s")
    wid = sid * 2 + cid
    for rr in range(2):
        row = wid * 2 + rr
        pltpu.sync_copy(cent_hbm.at[row], row_v)
        pat = _select_row(row_v, hist_v)
        thr_v[...] = lax.bitcast_convert_type(pat, jnp.float32)
        pltpu.sync_copy(thr_v, thr_hbm.at[row])


_MESH = plsc.VectorSubcoreMesh(
    core_axis_name="c", subcore_axis_name="s", num_cores=2, num_subcores=16)

_sc_select_call = functools.partial(
    pl.kernel,
    out_type=jax.ShapeDtypeStruct((BS, 16), jnp.float32),
    mesh=_MESH,
    scratch_types=[
        pltpu.VMEM((NQ,), jnp.float32),
        pltpu.VMEM((NBUCKET,), jnp.int32),
        pltpu.VMEM((16,), jnp.float32),
    ],
    compiler_params=pltpu.CompilerParams(needs_layout_passes=False),
)(_sc_select)


def kernel(bilinear_coords, boxes):
    bs, nq = bilinear_coords.shape[:2]
    x = bilinear_coords[:, :, 0]
    y = bilinear_coords[:, :, 1]
    bb = 8
    cent = pl.pallas_call(
        _cent_body,
        grid=(bs // bb,),
        in_specs=[
            pl.BlockSpec((bb, nq), lambda i: (i, 0)),
            pl.BlockSpec((bb, nq), lambda i: (i, 0)),
            pl.BlockSpec((bb, 4), lambda i: (i, 0)),
        ],
        out_specs=pl.BlockSpec((bb, nq), lambda i: (i, 0)),
        out_shape=jax.ShapeDtypeStruct((bs, nq), jnp.float32),
    )(x, y, boxes)

    thr16 = _sc_select_call(cent)

    mask = pl.pallas_call(
        _mask_body,
        grid=(bs // bb,),
        in_specs=[
            pl.BlockSpec((bb, nq), lambda i: (i, 0)),
            pl.BlockSpec((bb, 16), lambda i: (i, 0)),
        ],
        out_specs=pl.BlockSpec((bb, nq), lambda i: (i, 0)),
        out_shape=jax.ShapeDtypeStruct((bs, nq), jnp.bool_),
    )(cent, thr16)
    return cent, mask


# speedup vs baseline: 1.3805x; 1.3805x over previous
import functools

import jax
import jax.numpy as jnp
from jax import lax
from jax.experimental import pallas as pl
from jax.experimental.pallas import tpu as pltpu
from jax.experimental.pallas import tpu_sc as plsc

BS = 64
NQ = 32768
K = NQ // 16
NBUCKET = 1024
NB_VREG = NBUCKET // 16


def _cent_body(x_ref, y_ref, box_ref, cent_ref):
    xb = x_ref[...]
    yb = y_ref[...]
    cx = box_ref[:, 0:1]
    cy = box_ref[:, 1:2]
    w = box_ref[:, 2:3]
    h = box_ref[:, 3:4]
    xmin = cx - w / 2.0
    ymin = cy - h / 2.0
    xmax = cx + w / 2.0
    ymax = cy + h / 2.0
    left = jnp.clip(xb - xmin, 0.0, 1.0)
    right = jnp.clip(xmax - xb, 0.0, 1.0)
    top = jnp.clip(yb - ymin, 0.0, 1.0)
    down = jnp.clip(ymax - yb, 0.0, 1.0)
    sx = (left + right) / 2.0
    dx = jnp.abs(left - right) / 2.0
    sy = (top + down) / 2.0
    dy = jnp.abs(top - down) / 2.0
    cxn = (sx - dx) / (sx + dx)
    cyn = (sy - dy) / (sy + dy)
    cent_ref[...] = jnp.sqrt(cxn * cyn)


def _mask_body(cent_ref, thr_ref, mask_ref):
    mask_ref[...] = cent_ref[...] > thr_ref[:, 0:1]


def _select_row(row_v, hist_v):
    lanes = lax.iota(jnp.int32, 16)
    ones = jnp.ones((16,), jnp.int32)
    zeros16 = jnp.zeros((16,), jnp.int32)

    prefix = jnp.int32(0)
    need = jnp.int32(K + 1)
    for p in range(3):
        shift = 20 - 10 * p

        def zero_body(j, _):
            hist_v[pl.ds(j * 16, 16)] = zeros16
            return 0

        lax.fori_loop(0, NB_VREG, zero_body, 0, unroll=8)

        pfx = prefix

        def hist_body(i, _):
            v = row_v[pl.ds(i * 16, 16)]
            u = lax.bitcast_convert_type(v, jnp.int32)
            u = jnp.where(v == v, u, 0)
            bucket = lax.shift_right_logical(u, shift) & (NBUCKET - 1)
            m = u != 0
            if p != 0:
                m = jnp.logical_and(
                    m, lax.shift_right_logical(u, shift + 10) == pfx)
            plsc.addupdate_scatter(hist_v, [bucket], ones, mask=m)
            return 0

        lax.fori_loop(0, NQ // 16, hist_body, 0, unroll=4)

        def scan_cond(carry):
            t, _, found, _, _ = carry
            return jnp.logical_and(t < NB_VREG, jnp.logical_not(found))

        def scan_body(carry):
            t, above, found, bsel, above_sel = carry
            j = NB_VREG - 1 - t
            h = hist_v[pl.ds(j * 16, 16)]
            csum = plsc.cumsum(h)
            total = jnp.max(csum)
            s = total - csum + h
            ge = (above + s) >= need
            cnt = jnp.max(plsc.all_reduce_population_count(ge))
            istar = cnt - 1
            h_at = jnp.max(jnp.where(lanes == istar, h, 0))
            s_at = jnp.max(jnp.where(lanes == istar, s, jnp.int32(-2**31)))
            hit = cnt > 0
            bsel = jnp.where(hit, j * 16 + istar, bsel)
            above_sel = jnp.where(hit, above + s_at - h_at, above_sel)
            above = jnp.where(hit, above, above + total)
            return t + 1, above, hit, bsel, above_sel

        _, _, _, bsel, above_sel = lax.while_loop(
            scan_cond, scan_body,
            (jnp.int32(0), jnp.int32(0), jnp.bool_(False), jnp.int32(0),
             jnp.int32(0)))

        prefix = jnp.where(p == 0, bsel, (prefix << 10) | bsel)
        need = need - above_sel

    return jnp.broadcast_to(prefix << 0, (16,))


def _sc_select(cent_hbm, thr_hbm, row_v, hist_v, thr_v):
    cid = lax.axis_index("c")
    sid = lax.axis_index("s")
    wid = sid * 2 + cid
    for rr in range(2):
        row = wid * 2 + rr
        pltpu.sync_copy(cent_hbm.at[row], row_v)
        pat = _select_row(row_v, hist_v)
        thr_v[...] = lax.bitcast_convert_type(pat, jnp.float32)
        pltpu.sync_copy(thr_v, thr_hbm.at[row])


_MESH = plsc.VectorSubcoreMesh(
    core_axis_name="c", subcore_axis_name="s", num_cores=2, num_subcores=16)

_sc_select_call = functools.partial(
    pl.kernel,
    out_type=jax.ShapeDtypeStruct((BS, 16), jnp.float32),
    mesh=_MESH,
    scratch_types=[
        pltpu.VMEM((NQ,), jnp.float32),
        pltpu.VMEM((NBUCKET,), jnp.int32),
        pltpu.VMEM((16,), jnp.float32),
    ],
    compiler_params=pltpu.CompilerParams(needs_layout_passes=False),
)(_sc_select)


def kernel(bilinear_coords, boxes):
    bs, nq = bilinear_coords.shape[:2]
    x = bilinear_coords[:, :, 0]
    y = bilinear_coords[:, :, 1]
    bb = 8
    cent = pl.pallas_call(
        _cent_body,
        grid=(bs // bb,),
        in_specs=[
            pl.BlockSpec((bb, nq), lambda i: (i, 0)),
            pl.BlockSpec((bb, nq), lambda i: (i, 0)),
            pl.BlockSpec((bb, 4), lambda i: (i, 0)),
        ],
        out_specs=pl.BlockSpec((bb, nq), lambda i: (i, 0)),
        out_shape=jax.ShapeDtypeStruct((bs, nq), jnp.float32),
    )(x, y, boxes)

    thr16 = _sc_select_call(cent)

    mask = pl.pallas_call(
        _mask_body,
        grid=(bs // bb,),
        in_specs=[
            pl.BlockSpec((bb, nq), lambda i: (i, 0)),
            pl.BlockSpec((bb, 16), lambda i: (i, 0)),
        ],
        out_specs=pl.BlockSpec((bb, nq), lambda i: (i, 0)),
        out_shape=jax.ShapeDtypeStruct((bs, nq), jnp.bool_),
    )(cent, thr16)
    return cent, mask
